# baseline (speedup 1.0000x reference)
import functools

import jax
import jax.numpy as jnp
from jax import lax
from jax.experimental import pallas as pl
from jax.experimental.pallas import tpu as pltpu
from jax.experimental.pallas import tpu_sc as plsc

VOCAB = 1000000
EMBED_DIM = 32

NC = 2
NS = 16
NW = NC * NS

B_TOTAL = 16384 * 50
B_PER_W = B_TOTAL // NW
CHUNK = 1600
N_CHUNKS = B_PER_W // CHUNK
NSTREAM = 4
SUB = CHUNK // NSTREAM


def _gather_body(idx_hbm, table_hbm, out_hbm,
                 idx_v0, idx_v1, rows_v0, rows_v1,
                 sem_g0, sem_g1, sem_w0, sem_w1):
    wid = lax.axis_index("s") * NC + lax.axis_index("c")
    base = wid * B_PER_W

    idx_v = [idx_v0, idx_v1]
    rows_v = [rows_v0, rows_v1]
    sem_g = [sem_g0, sem_g1]
    sem_w = [sem_w0, sem_w1]

    def fire_gathers(b):
        return [
            pltpu.async_copy(
                table_hbm.at[idx_v[b].at[pl.ds(i * SUB, SUB)]],
                rows_v[b].at[pl.ds(i * SUB, SUB)],
                sem_g[b])
            for i in range(NSTREAM)
        ]

    pltpu.sync_copy(idx_hbm.at[pl.ds(base, CHUNK)], idx_v[0])
    gathers = [None] * N_CHUNKS
    writes = [None] * N_CHUNKS
    gathers[0] = fire_gathers(0)
    for j in range(N_CHUNKS):
        b = j % 2
        nb = 1 - b
        if j + 1 < N_CHUNKS:
            off_n = base + (j + 1) * CHUNK
            pltpu.sync_copy(idx_hbm.at[pl.ds(off_n, CHUNK)], idx_v[nb])
            if j >= 1:
                writes[j - 1].wait()
            gathers[j + 1] = fire_gathers(nb)
        for g in gathers[j]:
            g.wait()
        off = base + j * CHUNK
        writes[j] = pltpu.async_copy(
            rows_v[b], out_hbm.at[pl.ds(off, CHUNK)], sem_w[b])
    writes[N_CHUNKS - 2].wait()
    writes[N_CHUNKS - 1].wait()


@jax.jit
def _sc_gather(flat_idx, weight):
    mesh = plsc.VectorSubcoreMesh(core_axis_name="c", subcore_axis_name="s")
    return pl.kernel(
        _gather_body,
        out_type=jax.ShapeDtypeStruct((B_TOTAL, EMBED_DIM), jnp.float32),
        mesh=mesh,
        scratch_types=[
            pltpu.VMEM((CHUNK,), jnp.int32),
            pltpu.VMEM((CHUNK,), jnp.int32),
            pltpu.VMEM((CHUNK, EMBED_DIM), jnp.float32),
            pltpu.VMEM((CHUNK, EMBED_DIM), jnp.float32),
            pltpu.SemaphoreType.DMA,
            pltpu.SemaphoreType.DMA,
            pltpu.SemaphoreType.DMA,
            pltpu.SemaphoreType.DMA,
        ],
        compiler_params=pltpu.CompilerParams(use_tc_tiling_on_sc=False),
    )(flat_idx, weight)


def kernel(input_ids, weight):
    n, s = input_ids.shape
    flat_idx = input_ids.reshape(-1).astype(jnp.int32)
    out = _sc_gather(flat_idx, weight)
    return out.reshape(n, s, EMBED_DIM)


# speedup vs baseline: 1.2135x; 1.2135x over previous
import functools

import jax
import jax.numpy as jnp
from jax import lax
from jax.experimental import pallas as pl
from jax.experimental.pallas import tpu as pltpu
from jax.experimental.pallas import tpu_sc as plsc

VOCAB = 1000000
EMBED_DIM = 32
SEQ = 50
BATCH = 16384

NC = 2
NS = 16
NW = NC * NS

LANE = 16
NBLK = 128
N_CHUNKS_TOTAL = SEQ * (BATCH // NBLK)
CH_PER_W = N_CHUNKS_TOTAL // NW
J_PER_S = BATCH // NBLK


def _transpose_chunk(rows, tbuf):
    iota = lax.iota(jnp.int32, LANE)
    for d in range(EMBED_DIM):
        d_vec = jnp.full((LANE,), d, jnp.int32)
        for l0 in range(0, NBLK, LANE):
            v = plsc.load_gather(rows, [iota + l0, d_vec])
            tbuf[d // 8, d % 8, pl.ds(l0, LANE)] = v


def _gather_body(ids_hbm, table_hbm, out_hbm,
                 idx0, idx1, rows0, rows1, tb0, tb1,
                 sg0, sg1, sw0, sw1):
    wid = lax.axis_index("s") * NC + lax.axis_index("c")
    c0 = wid * CH_PER_W

    idx = [idx0, idx1]
    rows = [rows0, rows1]
    tb = [tb0, tb1]
    sg = [sg0, sg1]
    sw = [sw0, sw1]

    def chunk_sj(t):
        c = c0 + t
        return c // J_PER_S, lax.rem(c, J_PER_S)

    def start_gather(t, u):
        s, j = chunk_sj(t)
        pltpu.sync_copy(ids_hbm.at[s, pl.ds(j * NBLK, NBLK)], idx[u])
        pltpu.async_copy(table_hbm.at[idx[u]], rows[u], sg[u])

    def wait_gather(u):
        pltpu.make_async_copy(table_hbm.at[idx[u]], rows[u], sg[u]).wait()

    def start_write(t, u):
        s, j = chunk_sj(t)
        pltpu.async_copy(tb[u], out_hbm.at[s, :, j], sw[u])

    def wait_write(u):
        pltpu.make_async_copy(tb[u], out_hbm.at[0, :, 0], sw[u]).wait()

    def pair_body(t2, carry):
        for u in (0, 1):
            t = 2 * t2 + u

            @pl.when(t >= 2)
            def _():
                wait_write(u)

            @pl.when(jnp.logical_and(t >= 1, t <= CH_PER_W))
            def _():
                pu = 1 - u
                wait_gather(pu)
                _transpose_chunk(rows[pu], tb[pu])
                start_write(t - 1, pu)

            @pl.when(t < CH_PER_W)
            def _():
                start_gather(t, u)
        return carry

    lax.fori_loop(0, (CH_PER_W + 2 + 1) // 2, pair_body, 0)


@jax.jit
def _sc_gather(ids_t, weight):
    mesh = plsc.VectorSubcoreMesh(core_axis_name="c", subcore_axis_name="s")
    return pl.kernel(
        _gather_body,
        out_type=jax.ShapeDtypeStruct(
            (SEQ, EMBED_DIM // 8, J_PER_S, 8, NBLK), jnp.float32),
        mesh=mesh,
        scratch_types=[
            pltpu.VMEM((NBLK,), jnp.int32),
            pltpu.VMEM((NBLK,), jnp.int32),
            pltpu.VMEM((NBLK, EMBED_DIM), jnp.float32),
            pltpu.VMEM((NBLK, EMBED_DIM), jnp.float32),
            pltpu.VMEM((EMBED_DIM // 8, 8, NBLK), jnp.float32),
            pltpu.VMEM((EMBED_DIM // 8, 8, NBLK), jnp.float32),
            pltpu.SemaphoreType.DMA,
            pltpu.SemaphoreType.DMA,
            pltpu.SemaphoreType.DMA,
            pltpu.SemaphoreType.DMA,
        ],
        compiler_params=pltpu.CompilerParams(
            use_tc_tiling_on_sc=False, needs_layout_passes=False),
    )(ids_t, weight)


def kernel(input_ids, weight):
    ids_t = input_ids.astype(jnp.int32).T
    out5 = _sc_gather(ids_t, weight)
    return jnp.transpose(out5, (2, 4, 0, 1, 3)).reshape(BATCH, SEQ, EMBED_DIM)


# speedup vs baseline: 1.4800x; 1.2196x over previous
import functools

import jax
import jax.numpy as jnp
from jax import lax
from jax.experimental import pallas as pl
from jax.experimental.pallas import tpu as pltpu
from jax.experimental.pallas import tpu_sc as plsc

VOCAB = 1000000
EMBED_DIM = 32
SEQ = 50
BATCH = 16384

NC = 2
NS = 16
NW = NC * NS

LANE = 16
NBLK = 128
N_CHUNKS_TOTAL = SEQ * (BATCH // NBLK)
CH_PER_W = N_CHUNKS_TOTAL // NW
IDX_PER_W = CH_PER_W * NBLK
J_PER_S = BATCH // NBLK
ROW_W = 8 * NBLK
DEPTH = 4


def _transpose_chunk(rows, tbuf):
    iota = lax.iota(jnp.int32, LANE)

    def dbody(d, carry):
        dvec = jnp.full((LANE,), 0, jnp.int32) + d
        base = d * NBLK
        for l0 in range(0, NBLK, LANE):
            v = plsc.load_gather(rows, [iota + l0, dvec])
            tbuf[pl.ds(base + l0, LANE)] = v
        return carry

    lax.fori_loop(0, EMBED_DIM, dbody, 0)


def _gather_body(ids_hbm, table_hbm, out_hbm, idx_v, rows, tb, sg, sw):
    wid = lax.axis_index("s") * NC + lax.axis_index("c")
    c0 = wid * CH_PER_W

    pltpu.sync_copy(ids_hbm.at[pl.ds(c0 * NBLK, IDX_PER_W)], idx_v)

    def idx_slice(t):
        return idx_v.at[pl.ds(t * NBLK, NBLK)]

    def start_gather(t, u):
        pltpu.async_copy(table_hbm.at[idx_slice(t)], rows[u], sg[u])

    def wait_gather(t, u):
        pltpu.make_async_copy(table_hbm.at[idx_slice(t)], rows[u], sg[u]).wait()

    def start_write(t, u):
        c = c0 + t
        s = c // J_PER_S
        j = lax.rem(c, J_PER_S)
        for i in range(EMBED_DIM // 8):
            pltpu.async_copy(tb[u].at[pl.ds(i * ROW_W, ROW_W)],
                             out_hbm.at[(s * 4 + i) * J_PER_S + j], sw[u])

    def wait_write(u):
        for _ in range(EMBED_DIM // 8):
            pltpu.make_async_copy(tb[u].at[pl.ds(0, ROW_W)],
                                  out_hbm.at[0], sw[u]).wait()

    for t in range(DEPTH - 1):
        start_gather(t, t % DEPTH)

    def quad_body(q, carry):
        for u in range(DEPTH):
            t = DEPTH * q + u

            @pl.when(t < CH_PER_W)
            def _():
                @pl.when(t >= DEPTH)
                def _():
                    wait_write(u)
                wait_gather(t, u)
                _transpose_chunk(rows[u], tb[u])
                start_write(t, u)

            @pl.when(t + DEPTH - 1 < CH_PER_W)
            def _():
                start_gather(t + DEPTH - 1, (u + DEPTH - 1) % DEPTH)
        return carry

    lax.fori_loop(0, (CH_PER_W + DEPTH - 1) // DEPTH, quad_body, 0)
    for u in range(DEPTH):
        wait_write(u)


@jax.jit
def _sc_gather(ids_flat, weight):
    mesh = plsc.VectorSubcoreMesh(core_axis_name="c", subcore_axis_name="s")

    def body(ids_hbm, table_hbm, out_hbm,
             idx_v, r0, r1, r2, r3, t0, t1, t2, t3,
             g0, g1, g2, g3, w0, w1, w2, w3):
        _gather_body(ids_hbm, table_hbm, out_hbm, idx_v,
                     [r0, r1, r2, r3], [t0, t1, t2, t3],
                     [g0, g1, g2, g3], [w0, w1, w2, w3])

    return pl.kernel(
        body,
        out_type=jax.ShapeDtypeStruct(
            (SEQ * (EMBED_DIM // 8) * J_PER_S, ROW_W), jnp.float32),
        mesh=mesh,
        scratch_types=(
            [pltpu.VMEM((IDX_PER_W,), jnp.int32)]
            + [pltpu.VMEM((NBLK, EMBED_DIM), jnp.float32)] * DEPTH
            + [pltpu.VMEM((EMBED_DIM * NBLK,), jnp.float32)] * DEPTH
            + [pltpu.SemaphoreType.DMA] * (2 * DEPTH)
        ),
        compiler_params=pltpu.CompilerParams(
            use_tc_tiling_on_sc=False, needs_layout_passes=False),
    )(ids_flat, weight)


def kernel(input_ids, weight):
    ids_flat = input_ids.astype(jnp.int32).T.reshape(-1)
    out2 = _sc_gather(ids_flat, weight)
    out5 = out2.reshape(SEQ, EMBED_DIM // 8, J_PER_S, 8, NBLK)
    return jnp.transpose(out5, (2, 4, 0, 1, 3)).reshape(BATCH, SEQ, EMBED_DIM)
